# baseline (speedup 1.0000x reference)
import functools

import jax
import jax.numpy as jnp
from jax import lax
from jax.experimental import pallas as pl
from jax.experimental.pallas import tpu as pltpu
from jax.experimental.pallas import tpu_sc as plsc

C, T, H, W = 3, 64, 384, 384
TS = T // 4
PPF = 6
QROWS = H // PPF
NW = 32
PER_W = C * T * PPF // NW
NBUF = 5


def _sc_fast_copy(frames):
    mesh = plsc.VectorSubcoreMesh(core_axis_name="c", subcore_axis_name="s")

    @functools.partial(
        pl.kernel,
        mesh=mesh,
        out_type=jax.ShapeDtypeStruct((C, T, H, W), jnp.float32),
        scratch_types=[
            pltpu.VMEM((NBUF, QROWS, W), jnp.float32),
            pltpu.SemaphoreType.DMA,
            pltpu.SemaphoreType.DMA,
        ],
        compiler_params=pltpu.CompilerParams(use_tc_tiling_on_sc=True),
    )
    def k(src, fast_out, buf, sem_r, sem_w):
        wid = lax.axis_index("s") * 2 + lax.axis_index("c")

        def coords(j):
            p = wid * PER_W + j
            return p // (T * PPF), (p // PPF) % T, p % PPF

        def rd(j):
            c, t, q = coords(j)
            rows = pl.ds(q * QROWS, QROWS)
            return pltpu.make_async_copy(
                src.at[c, t, rows], buf.at[j % NBUF], sem_r
            )

        def wr(j):
            c, t, q = coords(j)
            rows = pl.ds(q * QROWS, QROWS)
            return pltpu.make_async_copy(
                buf.at[j % NBUF], fast_out.at[c, t, rows], sem_w
            )

        rd(0).start()
        rd(1).start()
        for j in range(PER_W):
            rd(j).wait()
            if j >= 3:
                wr(j - 3).wait()
            wr(j).start()
            if j + 2 < PER_W:
                rd(j + 2).start()
        for j in range(max(0, PER_W - 3), PER_W):
            wr(j).wait()

    return k(frames)


def _tc_slow_gather(frames):
    def body(src_ref, out_ref, sem):
        copies = [
            pltpu.make_async_copy(
                src_ref.at[c, (i * 21) // 5], out_ref.at[c, i], sem
            )
            for c in range(C)
            for i in range(TS)
        ]
        for cp in copies:
            cp.start()
        for cp in copies:
            cp.wait()

    return pl.pallas_call(
        body,
        in_specs=[pl.BlockSpec(memory_space=pl.ANY)],
        out_specs=pl.BlockSpec(memory_space=pl.ANY),
        out_shape=jax.ShapeDtypeStruct((C, TS, H, W), jnp.float32),
        scratch_shapes=[pltpu.SemaphoreType.DMA],
    )(frames)


def kernel(frames):
    fast = _sc_fast_copy(frames)
    slow = _tc_slow_gather(frames)
    return (slow, fast)


# speedup vs baseline: 7.7296x; 7.7296x over previous
import functools

import jax
import jax.numpy as jnp
from jax import lax
from jax.experimental import pallas as pl
from jax.experimental.pallas import tpu as pltpu
from jax.experimental.pallas import tpu_sc as plsc

C, T, H, W = 3, 64, 384, 384
TS = T // 4
PPF = 6
QROWS = H // PPF
NW = 32
PER_W = C * T * PPF // NW
NBUF = 5


def _sc_fast_copy(frames):
    mesh = plsc.VectorSubcoreMesh(core_axis_name="c", subcore_axis_name="s")

    @functools.partial(
        pl.kernel,
        mesh=mesh,
        out_type=jax.ShapeDtypeStruct((C, T, H, W), jnp.float32),
        scratch_types=[
            pltpu.VMEM((NBUF, QROWS, W), jnp.float32),
            pltpu.SemaphoreType.DMA,
            pltpu.SemaphoreType.DMA,
        ],
        compiler_params=pltpu.CompilerParams(use_tc_tiling_on_sc=True),
    )
    def k(src, fast_out, buf, sem_r, sem_w):
        wid = lax.axis_index("s") * 2 + lax.axis_index("c")

        def coords(j):
            p = wid * PER_W + j
            return p // (T * PPF), (p // PPF) % T, p % PPF

        def rd(j):
            c, t, q = coords(j)
            rows = pl.ds(q * QROWS, QROWS)
            return pltpu.make_async_copy(
                src.at[c, t, rows], buf.at[j % NBUF], sem_r
            )

        def wr(j):
            c, t, q = coords(j)
            rows = pl.ds(q * QROWS, QROWS)
            return pltpu.make_async_copy(
                buf.at[j % NBUF], fast_out.at[c, t, rows], sem_w
            )

        rd(0).start()
        rd(1).start()
        for j in range(PER_W):
            rd(j).wait()
            if j >= 3:
                wr(j - 3).wait()
            wr(j).start()
            if j + 2 < PER_W:
                rd(j + 2).start()
        for j in range(max(0, PER_W - 3), PER_W):
            wr(j).wait()

    return k(frames)


def _tc_slow_gather(frames):
    frames_list = [(c, (i * 21) // 5, i) for c in range(C) for i in range(TS)]
    n = len(frames_list)

    def body(src_ref, out_ref, buf, sem_r, sem_w):
        def rd(k):
            c, t, _ = frames_list[k]
            return pltpu.make_async_copy(
                src_ref.at[c, t], buf.at[k % 4], sem_r
            )

        def wr(k):
            c, _, i = frames_list[k]
            return pltpu.make_async_copy(
                buf.at[k % 4], out_ref.at[c, i], sem_w
            )

        rd(0).start()
        rd(1).start()
        for k in range(n):
            rd(k).wait()
            if k >= 2:
                wr(k - 2).wait()
            wr(k).start()
            if k + 2 < n:
                rd(k + 2).start()
        wr(n - 2).wait()
        wr(n - 1).wait()

    return pl.pallas_call(
        body,
        in_specs=[pl.BlockSpec(memory_space=pl.ANY)],
        out_specs=pl.BlockSpec(memory_space=pl.ANY),
        out_shape=jax.ShapeDtypeStruct((C, TS, H, W), jnp.float32),
        scratch_shapes=[
            pltpu.VMEM((4, H, W), jnp.float32),
            pltpu.SemaphoreType.DMA,
            pltpu.SemaphoreType.DMA,
        ],
    )(frames)


def kernel(frames):
    fast = _sc_fast_copy(frames)
    slow = _tc_slow_gather(frames)
    return (slow, fast)


# speedup vs baseline: 8.0448x; 1.0408x over previous
import functools

import jax
import jax.numpy as jnp
from jax import lax
from jax.experimental import pallas as pl
from jax.experimental.pallas import tpu as pltpu
from jax.experimental.pallas import tpu_sc as plsc

C, T, H, W = 3, 64, 384, 384
TS = T // 4
PPF = 6
QROWS = H // PPF
NW = 32
PER_W = C * TS * PPF // NW
NBUF = 5


def _sc_slow_gather(frames):
    mesh = plsc.VectorSubcoreMesh(core_axis_name="c", subcore_axis_name="s")

    @functools.partial(
        pl.kernel,
        mesh=mesh,
        out_type=jax.ShapeDtypeStruct((C, TS, H, W), jnp.float32),
        scratch_types=[
            pltpu.VMEM((NBUF, QROWS, W), jnp.float32),
            pltpu.SemaphoreType.DMA((NBUF,)),
            pltpu.SemaphoreType.DMA((NBUF,)),
        ],
        compiler_params=pltpu.CompilerParams(use_tc_tiling_on_sc=True),
    )
    def k(src, slow_out, buf, sem_r, sem_w):
        wid = lax.axis_index("s") * 2 + lax.axis_index("c")

        def coords(j):
            p = wid * PER_W + j
            c = p // (TS * PPF)
            i = (p // PPF) % TS
            q = p % PPF
            return c, i, q

        def rd(j):
            c, i, q = coords(j)
            t = (i * 21) // 5
            rows = pl.ds(q * QROWS, QROWS)
            return pltpu.make_async_copy(
                src.at[c, t, rows], buf.at[j % NBUF], sem_r.at[j % NBUF]
            )

        def wr(j):
            c, i, q = coords(j)
            rows = pl.ds(q * QROWS, QROWS)
            return pltpu.make_async_copy(
                buf.at[j % NBUF], slow_out.at[c, i, rows], sem_w.at[j % NBUF]
            )

        rd(0).start()
        rd(1).start()
        for j in range(PER_W):
            rd(j).wait()
            if j >= 3:
                wr(j - 3).wait()
            wr(j).start()
            if j + 2 < PER_W:
                rd(j + 2).start()
        for j in range(max(0, PER_W - 3), PER_W):
            wr(j).wait()

    return k(frames)


def _tc_fast_copy(frames):
    seq = [(c, t) for c in range(C) for t in range(T)]
    n = len(seq)
    NB = 8
    LOOK = 6

    def body(src_ref, out_ref, buf, sem_r, sem_w):
        def rd(k):
            c, t = seq[k]
            return pltpu.make_async_copy(
                src_ref.at[c, t], buf.at[k % NB], sem_r.at[k % NB]
            )

        def wr(k):
            c, t = seq[k]
            return pltpu.make_async_copy(
                buf.at[k % NB], out_ref.at[c, t], sem_w.at[k % NB]
            )

        for k in range(LOOK):
            rd(k).start()
        for k in range(n):
            rd(k).wait()
            wr(k).start()
            if k + LOOK < n:
                if k + LOOK >= NB:
                    wr(k + LOOK - NB).wait()
                rd(k + LOOK).start()
        for k in range(n - NB, n):
            wr(k).wait()

    return pl.pallas_call(
        body,
        in_specs=[pl.BlockSpec(memory_space=pl.ANY)],
        out_specs=pl.BlockSpec(memory_space=pl.ANY),
        out_shape=jax.ShapeDtypeStruct((C, T, H, W), jnp.float32),
        scratch_shapes=[
            pltpu.VMEM((NB, H, W), jnp.float32),
            pltpu.SemaphoreType.DMA((NB,)),
            pltpu.SemaphoreType.DMA((NB,)),
        ],
    )(frames)


def kernel(frames):
    slow = _sc_slow_gather(frames)
    fast = _tc_fast_copy(frames)
    return (slow, fast)
